# Initial kernel scaffold; baseline (speedup 1.0000x reference)
import jax
import jax.numpy as jnp
from jax import lax
from jax.experimental import pallas as pl
from jax.experimental.pallas import tpu as pltpu
from jax.experimental.pallas import tpu_sc as plsc

N = 10000
E = 320000
D_IN = 128
HID = 256
NW = 32
E_PER_W = E // NW
L = 16


def _sc_hist_body(src_hbm, dst_hbm, agent_hbm, deg_out, c_out,
                  src_v, dst_v, agent_v, deg_v, c_v):
    wid = lax.axis_index("s") * 2 + lax.axis_index("c")

    pltpu.sync_copy(src_hbm.at[wid], src_v)
    pltpu.sync_copy(dst_hbm.at[wid], dst_v)
    pltpu.sync_copy(agent_hbm, agent_v)

    zero = jnp.zeros((L,), jnp.float32)

    def _zero(i, _):
        deg_v[pl.ds(i * L, L)] = zero
        c_v[pl.ds(i * L, L)] = zero
        return 0

    lax.fori_loop(0, N // L, _zero, 0, unroll=4)

    agent = agent_v[...]
    ones = jnp.ones((L,), jnp.float32)

    def _scan(i, _):
        s = src_v[pl.ds(i * L, L)]
        d = dst_v[pl.ds(i * L, L)]
        plsc.addupdate_scatter(deg_v, [d], ones)
        plsc.addupdate_scatter(c_v, [s], ones, mask=d == agent)
        return 0

    lax.fori_loop(0, E_PER_W // L, _scan, 0, unroll=4)

    pltpu.sync_copy(deg_v, deg_out.at[wid])
    pltpu.sync_copy(c_v, c_out.at[wid])


_sc_hist = pl.kernel(
    _sc_hist_body,
    out_type=(
        jax.ShapeDtypeStruct((NW, N), jnp.float32),
        jax.ShapeDtypeStruct((NW, N), jnp.float32),
    ),
    mesh=plsc.VectorSubcoreMesh(core_axis_name="c", subcore_axis_name="s"),
    scratch_types=[
        pltpu.VMEM((E_PER_W,), jnp.int32),
        pltpu.VMEM((E_PER_W,), jnp.int32),
        pltpu.VMEM((L,), jnp.int32),
        pltpu.VMEM((N,), jnp.float32),
        pltpu.VMEM((N,), jnp.float32),
    ],
)


def _tc_head_body(agent_ref, state_ref, degp_ref, cp_ref,
                  Wc_ref, bc_ref, W1_ref, b1_ref, g1_ref, bt1_ref,
                  W2_ref, b2_ref, g2_ref, bt2_ref, Wmu_ref, bmu_ref, out_ref):
    a = agent_ref[0]
    deg = jnp.sum(degp_ref[...], axis=0, keepdims=True) + 1.0
    c = jnp.sum(cp_ref[...], axis=0, keepdims=True)
    dinv = lax.rsqrt(deg)
    col = lax.broadcasted_iota(jnp.int32, (1, N), 1)
    is_a = col == a
    da = jnp.sum(jnp.where(is_a, dinv, 0.0))
    u = da * (c * dinv) + (da * da) * jnp.where(is_a, 1.0, 0.0)

    acc = jnp.dot(u, state_ref[...], preferred_element_type=jnp.float32)
    x = jnp.dot(acc, Wc_ref[...], preferred_element_type=jnp.float32) + bc_ref[...]
    x = jnp.maximum(x, 0.0)

    x = jnp.dot(x, W1_ref[...], preferred_element_type=jnp.float32) + b1_ref[...]
    m = jnp.mean(x, axis=-1, keepdims=True)
    v = jnp.mean((x - m) ** 2, axis=-1, keepdims=True)
    x = (x - m) * lax.rsqrt(v + 1e-5) * g1_ref[...] + bt1_ref[...]
    x = jnp.maximum(x, 0.0)

    x = jnp.dot(x, W2_ref[...], preferred_element_type=jnp.float32) + b2_ref[...]
    m = jnp.mean(x, axis=-1, keepdims=True)
    v = jnp.mean((x - m) ** 2, axis=-1, keepdims=True)
    x = (x - m) * lax.rsqrt(v + 1e-5) * g2_ref[...] + bt2_ref[...]
    x = jnp.maximum(x, 0.0)

    x = jnp.dot(x, Wmu_ref[...], preferred_element_type=jnp.float32) + bmu_ref[...]
    out_ref[...] = jax.nn.sigmoid(x)


def _tc_head(agent, state, deg_parts, c_parts, Wc, bc, W1, b1, g1, bt1,
             W2, b2, g2, bt2, Wmu, bmu):
    vspec = pl.BlockSpec()
    return pl.pallas_call(
        _tc_head_body,
        out_shape=jax.ShapeDtypeStruct((1, 64), jnp.float32),
        in_specs=[pl.BlockSpec(memory_space=pltpu.SMEM)] + [vspec] * 15,
        out_specs=vspec,
    )(agent, state, deg_parts, c_parts, Wc, bc, W1, b1, g1, bt1,
      W2, b2, g2, bt2, Wmu, bmu)


def kernel(state, edge_index, agent_i, W_conv, b_conv, W1, b1, g1, beta1,
           W2, b2, g2, beta2, Wmu, bmu):
    ei = edge_index.astype(jnp.int32)
    src = ei[0].reshape(NW, E_PER_W)
    dst = ei[1].reshape(NW, E_PER_W)
    agent_vec = jnp.full((L,), agent_i, dtype=jnp.int32)
    deg_parts, c_parts = _sc_hist(src, dst, agent_vec)

    agent = jnp.asarray(agent_i, jnp.int32).reshape(1)
    out = _tc_head(agent, state, deg_parts, c_parts,
                   W_conv, b_conv.reshape(1, HID),
                   W1, b1.reshape(1, 256), g1.reshape(1, 256), beta1.reshape(1, 256),
                   W2, b2.reshape(1, 128), g2.reshape(1, 128), beta2.reshape(1, 128),
                   Wmu, bmu.reshape(1, 64))
    return out.reshape(64)


# speedup vs baseline: 179.3385x; 179.3385x over previous
import jax
import jax.numpy as jnp
from jax import lax
from jax.experimental import pallas as pl
from jax.experimental.pallas import tpu as pltpu
from jax.experimental.pallas import tpu_sc as plsc

N = 10000
E = 320000
D_IN = 128
HID = 256
NW = 32
E_PER_W = E // NW
L = 16


def _sc_hist_body(src_hbm, dst_hbm, agent_hbm, deg_out, c_out,
                  src_v, dst_v, agent_v, deg_v, c_v):
    wid = lax.axis_index("s") * 2 + lax.axis_index("c")

    pltpu.sync_copy(src_hbm.at[wid], src_v)
    pltpu.sync_copy(dst_hbm.at[wid], dst_v)
    pltpu.sync_copy(agent_hbm, agent_v)

    zero = jnp.zeros((L,), jnp.float32)

    def _zero(i, _):
        deg_v[pl.ds(i * L, L)] = zero
        c_v[pl.ds(i * L, L)] = zero
        return 0

    lax.fori_loop(0, N // L, _zero, 0, unroll=4)

    agent = agent_v[...]
    ones = jnp.ones((L,), jnp.float32)

    def _scan(i, _):
        s = src_v[pl.ds(i * L, L)]
        d = dst_v[pl.ds(i * L, L)]
        plsc.addupdate_scatter(deg_v, [d], ones)
        plsc.addupdate_scatter(c_v, [s], ones, mask=d == agent)
        return 0

    lax.fori_loop(0, E_PER_W // L, _scan, 0, unroll=4)

    pltpu.sync_copy(deg_v, deg_out.at[wid])
    pltpu.sync_copy(c_v, c_out.at[wid])


_sc_hist = pl.kernel(
    _sc_hist_body,
    out_type=(
        jax.ShapeDtypeStruct((NW, N), jnp.float32),
        jax.ShapeDtypeStruct((NW, N), jnp.float32),
    ),
    mesh=plsc.VectorSubcoreMesh(core_axis_name="c", subcore_axis_name="s"),
    compiler_params=pltpu.CompilerParams(needs_layout_passes=False),
    scratch_types=[
        pltpu.VMEM((E_PER_W,), jnp.int32),
        pltpu.VMEM((E_PER_W,), jnp.int32),
        pltpu.VMEM((L,), jnp.int32),
        pltpu.VMEM((N,), jnp.float32),
        pltpu.VMEM((N,), jnp.float32),
    ],
)


def _tc_head_body(agent_ref, state_ref, degp_ref, cp_ref,
                  Wc_ref, bc_ref, W1_ref, b1_ref, g1_ref, bt1_ref,
                  W2_ref, b2_ref, g2_ref, bt2_ref, Wmu_ref, bmu_ref, out_ref):
    a = agent_ref[0]
    deg = jnp.sum(degp_ref[...], axis=0, keepdims=True) + 1.0
    c = jnp.sum(cp_ref[...], axis=0, keepdims=True)
    dinv = lax.rsqrt(deg)
    col = lax.broadcasted_iota(jnp.int32, (1, N), 1)
    is_a = col == a
    da = jnp.sum(jnp.where(is_a, dinv, 0.0))
    u = da * (c * dinv) + (da * da) * jnp.where(is_a, 1.0, 0.0)

    acc = jnp.dot(u, state_ref[...], preferred_element_type=jnp.float32)
    x = jnp.dot(acc, Wc_ref[...], preferred_element_type=jnp.float32) + bc_ref[...]
    x = jnp.maximum(x, 0.0)

    x = jnp.dot(x, W1_ref[...], preferred_element_type=jnp.float32) + b1_ref[...]
    m = jnp.mean(x, axis=-1, keepdims=True)
    v = jnp.mean((x - m) ** 2, axis=-1, keepdims=True)
    x = (x - m) * lax.rsqrt(v + 1e-5) * g1_ref[...] + bt1_ref[...]
    x = jnp.maximum(x, 0.0)

    x = jnp.dot(x, W2_ref[...], preferred_element_type=jnp.float32) + b2_ref[...]
    m = jnp.mean(x, axis=-1, keepdims=True)
    v = jnp.mean((x - m) ** 2, axis=-1, keepdims=True)
    x = (x - m) * lax.rsqrt(v + 1e-5) * g2_ref[...] + bt2_ref[...]
    x = jnp.maximum(x, 0.0)

    x = jnp.dot(x, Wmu_ref[...], preferred_element_type=jnp.float32) + bmu_ref[...]
    out_ref[...] = jax.nn.sigmoid(x)


def _tc_head(agent, state, deg_parts, c_parts, Wc, bc, W1, b1, g1, bt1,
             W2, b2, g2, bt2, Wmu, bmu):
    vspec = pl.BlockSpec()
    return pl.pallas_call(
        _tc_head_body,
        out_shape=jax.ShapeDtypeStruct((1, 64), jnp.float32),
        in_specs=[pl.BlockSpec(memory_space=pltpu.SMEM)] + [vspec] * 15,
        out_specs=vspec,
    )(agent, state, deg_parts, c_parts, Wc, bc, W1, b1, g1, bt1,
      W2, b2, g2, bt2, Wmu, bmu)


def kernel(state, edge_index, agent_i, W_conv, b_conv, W1, b1, g1, beta1,
           W2, b2, g2, beta2, Wmu, bmu):
    ei = edge_index.astype(jnp.int32)
    src = ei[0].reshape(NW, E_PER_W)
    dst = ei[1].reshape(NW, E_PER_W)
    agent_vec = jnp.full((L,), agent_i, dtype=jnp.int32)
    deg_parts, c_parts = _sc_hist(src, dst, agent_vec)

    agent = jnp.asarray(agent_i, jnp.int32).reshape(1)
    out = _tc_head(agent, state, deg_parts, c_parts,
                   W_conv, b_conv.reshape(1, HID),
                   W1, b1.reshape(1, 256), g1.reshape(1, 256), beta1.reshape(1, 256),
                   W2, b2.reshape(1, 128), g2.reshape(1, 128), beta2.reshape(1, 128),
                   Wmu, bmu.reshape(1, 64))
    return out.reshape(64)


# speedup vs baseline: 235.9237x; 1.3155x over previous
import jax
import jax.numpy as jnp
from jax import lax
from jax.experimental import pallas as pl
from jax.experimental.pallas import tpu as pltpu
from jax.experimental.pallas import tpu_sc as plsc

N = 10000
E = 320000
D_IN = 128
HID = 256
NW = 32
E_PER_W = E // NW
L = 16


def _sc_hist_body(edge_hbm, agent_hbm, deg_out, c_out,
                  src_v, dst_v, agent_v, deg_v, c_v):
    wid = lax.axis_index("s") * 2 + lax.axis_index("c")

    pltpu.sync_copy(edge_hbm.at[pl.ds(wid * E_PER_W, E_PER_W)], src_v)
    pltpu.sync_copy(edge_hbm.at[pl.ds(E + wid * E_PER_W, E_PER_W)], dst_v)
    pltpu.sync_copy(agent_hbm, agent_v)

    zero = jnp.zeros((L,), jnp.float32)

    def _zero(i, _):
        deg_v[pl.ds(i * L, L)] = zero
        c_v[pl.ds(i * L, L)] = zero
        return 0

    lax.fori_loop(0, N // L, _zero, 0, unroll=4)

    agent = agent_v[...]
    ones = jnp.ones((L,), jnp.float32)

    def _scan(i, _):
        s = src_v[pl.ds(i * L, L)]
        d = dst_v[pl.ds(i * L, L)]
        plsc.addupdate_scatter(deg_v, [d], ones)
        plsc.addupdate_scatter(c_v, [s], ones, mask=d == agent)
        return 0

    lax.fori_loop(0, E_PER_W // L, _scan, 0, unroll=4)

    pltpu.sync_copy(deg_v, deg_out.at[wid])
    pltpu.sync_copy(c_v, c_out.at[wid])


_sc_hist = pl.kernel(
    _sc_hist_body,
    out_type=(
        jax.ShapeDtypeStruct((NW, N), jnp.float32),
        jax.ShapeDtypeStruct((NW, N), jnp.float32),
    ),
    mesh=plsc.VectorSubcoreMesh(core_axis_name="c", subcore_axis_name="s"),
    compiler_params=pltpu.CompilerParams(needs_layout_passes=False),
    scratch_types=[
        pltpu.VMEM((E_PER_W,), jnp.int32),
        pltpu.VMEM((E_PER_W,), jnp.int32),
        pltpu.VMEM((L,), jnp.int32),
        pltpu.VMEM((N,), jnp.float32),
        pltpu.VMEM((N,), jnp.float32),
    ],
)


def _tc_head_body(agent_ref, state_ref, degp_ref, cp_ref,
                  Wc_ref, bc_ref, W1_ref, b1_ref, g1_ref, bt1_ref,
                  W2_ref, b2_ref, g2_ref, bt2_ref, Wmu_ref, bmu_ref, out_ref):
    a = agent_ref[0]
    deg = jnp.sum(degp_ref[...], axis=0, keepdims=True) + 1.0
    c = jnp.sum(cp_ref[...], axis=0, keepdims=True)
    dinv = lax.rsqrt(deg)
    col = lax.broadcasted_iota(jnp.int32, (1, N), 1)
    is_a = col == a
    da = jnp.sum(jnp.where(is_a, dinv, 0.0))
    u = da * (c * dinv) + (da * da) * jnp.where(is_a, 1.0, 0.0)

    acc = jnp.dot(u, state_ref[...], preferred_element_type=jnp.float32)
    x = jnp.dot(acc, Wc_ref[...], preferred_element_type=jnp.float32) + bc_ref[...]
    x = jnp.maximum(x, 0.0)

    x = jnp.dot(x, W1_ref[...], preferred_element_type=jnp.float32) + b1_ref[...]
    m = jnp.mean(x, axis=-1, keepdims=True)
    v = jnp.mean((x - m) ** 2, axis=-1, keepdims=True)
    x = (x - m) * lax.rsqrt(v + 1e-5) * g1_ref[...] + bt1_ref[...]
    x = jnp.maximum(x, 0.0)

    x = jnp.dot(x, W2_ref[...], preferred_element_type=jnp.float32) + b2_ref[...]
    m = jnp.mean(x, axis=-1, keepdims=True)
    v = jnp.mean((x - m) ** 2, axis=-1, keepdims=True)
    x = (x - m) * lax.rsqrt(v + 1e-5) * g2_ref[...] + bt2_ref[...]
    x = jnp.maximum(x, 0.0)

    x = jnp.dot(x, Wmu_ref[...], preferred_element_type=jnp.float32) + bmu_ref[...]
    out_ref[...] = jax.nn.sigmoid(x)


def _tc_head(agent, state, deg_parts, c_parts, Wc, bc, W1, b1, g1, bt1,
             W2, b2, g2, bt2, Wmu, bmu):
    vspec = pl.BlockSpec()
    return pl.pallas_call(
        _tc_head_body,
        out_shape=jax.ShapeDtypeStruct((1, 64), jnp.float32),
        in_specs=[pl.BlockSpec(memory_space=pltpu.SMEM)] + [vspec] * 15,
        out_specs=vspec,
    )(agent, state, deg_parts, c_parts, Wc, bc, W1, b1, g1, bt1,
      W2, b2, g2, bt2, Wmu, bmu)


def kernel(state, edge_index, agent_i, W_conv, b_conv, W1, b1, g1, beta1,
           W2, b2, g2, beta2, Wmu, bmu):
    ei = edge_index.astype(jnp.int32).reshape(2 * E)
    agent_vec = jnp.full((L,), agent_i, dtype=jnp.int32)
    deg_parts, c_parts = _sc_hist(ei, agent_vec)

    agent = jnp.asarray(agent_i, jnp.int32).reshape(1)
    out = _tc_head(agent, state, deg_parts, c_parts,
                   W_conv, b_conv.reshape(1, HID),
                   W1, b1.reshape(1, 256), g1.reshape(1, 256), beta1.reshape(1, 256),
                   W2, b2.reshape(1, 128), g2.reshape(1, 128), beta2.reshape(1, 128),
                   Wmu, bmu.reshape(1, 64))
    return out.reshape(64)


# speedup vs baseline: 263.9588x; 1.1188x over previous
import jax
import jax.numpy as jnp
from jax import lax
from jax.experimental import pallas as pl
from jax.experimental.pallas import tpu as pltpu
from jax.experimental.pallas import tpu_sc as plsc

N = 10000
E = 320000
D_IN = 128
HID = 256
NW = 32
E_PER_W = E // NW
L = 16


_BLKS = E // 128
_B_LO = _BLKS // NW
_B_EXTRA = _BLKS % NW
_CHUNK_MAX = (_B_LO + 1) * 128


def _sc_hist_body(edge_hbm, agent_hbm, deg_out, c_out,
                  ev, agent_v, deg_v, c_v):
    wid = lax.axis_index("s") * 2 + lax.axis_index("c")

    nblk = _B_LO + jnp.where(wid < _B_EXTRA, 1, 0)
    col0 = wid * (_B_LO * 128) + jnp.minimum(wid, _B_EXTRA) * 128
    ncol = nblk * 128

    pltpu.sync_copy(edge_hbm.at[:, pl.ds(col0, ncol)], ev.at[:, pl.ds(0, ncol)])
    pltpu.sync_copy(agent_hbm, agent_v)

    zero = jnp.zeros((L,), jnp.float32)

    def _zero(i, _):
        deg_v[pl.ds(i * L, L)] = zero
        c_v[pl.ds(i * L, L)] = zero
        return 0

    lax.fori_loop(0, N // L, _zero, 0, unroll=4)

    agent = agent_v[...]
    ones = jnp.ones((L,), jnp.float32)

    def _scan(i, _):
        s = ev[0, pl.ds(i * L, L)]
        d = ev[1, pl.ds(i * L, L)]
        plsc.addupdate_scatter(deg_v, [d], ones)
        plsc.addupdate_scatter(c_v, [s], ones, mask=d == agent)
        return 0

    lax.fori_loop(0, _B_LO * 128 // L, _scan, 0, unroll=4)

    @pl.when(wid < _B_EXTRA)
    def _extra():
        lax.fori_loop(_B_LO * 128 // L, (_B_LO + 1) * 128 // L, _scan, 0,
                      unroll=4)

    pltpu.sync_copy(deg_v, deg_out.at[wid])
    pltpu.sync_copy(c_v, c_out.at[wid])


_sc_hist = pl.kernel(
    _sc_hist_body,
    out_type=(
        jax.ShapeDtypeStruct((NW, N), jnp.float32),
        jax.ShapeDtypeStruct((NW, N), jnp.float32),
    ),
    mesh=plsc.VectorSubcoreMesh(core_axis_name="c", subcore_axis_name="s"),
    compiler_params=pltpu.CompilerParams(needs_layout_passes=False),
    scratch_types=[
        pltpu.VMEM((2, _CHUNK_MAX), jnp.int32),
        pltpu.VMEM((L,), jnp.int32),
        pltpu.VMEM((N,), jnp.float32),
        pltpu.VMEM((N,), jnp.float32),
    ],
)


def _tc_head_body(agent_ref, state_ref, degp_ref, cp_ref,
                  Wc_ref, bc_ref, W1_ref, b1_ref, g1_ref, bt1_ref,
                  W2_ref, b2_ref, g2_ref, bt2_ref, Wmu_ref, bmu_ref, out_ref):
    a = agent_ref[0]
    deg = jnp.sum(degp_ref[...], axis=0, keepdims=True) + 1.0
    c = jnp.sum(cp_ref[...], axis=0, keepdims=True)
    dinv = lax.rsqrt(deg)
    col = lax.broadcasted_iota(jnp.int32, (1, N), 1)
    is_a = col == a
    da = jnp.sum(jnp.where(is_a, dinv, 0.0))
    u = da * (c * dinv) + (da * da) * jnp.where(is_a, 1.0, 0.0)

    acc = jnp.dot(u, state_ref[...], preferred_element_type=jnp.float32)
    x = jnp.dot(acc, Wc_ref[...], preferred_element_type=jnp.float32) + bc_ref[...]
    x = jnp.maximum(x, 0.0)

    x = jnp.dot(x, W1_ref[...], preferred_element_type=jnp.float32) + b1_ref[...]
    m = jnp.mean(x, axis=-1, keepdims=True)
    v = jnp.mean((x - m) ** 2, axis=-1, keepdims=True)
    x = (x - m) * lax.rsqrt(v + 1e-5) * g1_ref[...] + bt1_ref[...]
    x = jnp.maximum(x, 0.0)

    x = jnp.dot(x, W2_ref[...], preferred_element_type=jnp.float32) + b2_ref[...]
    m = jnp.mean(x, axis=-1, keepdims=True)
    v = jnp.mean((x - m) ** 2, axis=-1, keepdims=True)
    x = (x - m) * lax.rsqrt(v + 1e-5) * g2_ref[...] + bt2_ref[...]
    x = jnp.maximum(x, 0.0)

    x = jnp.dot(x, Wmu_ref[...], preferred_element_type=jnp.float32) + bmu_ref[...]
    out_ref[...] = jax.nn.sigmoid(x)


def _tc_head(agent, state, deg_parts, c_parts, Wc, bc, W1, b1, g1, bt1,
             W2, b2, g2, bt2, Wmu, bmu):
    vspec = pl.BlockSpec()
    return pl.pallas_call(
        _tc_head_body,
        out_shape=jax.ShapeDtypeStruct((1, 64), jnp.float32),
        in_specs=[pl.BlockSpec(memory_space=pltpu.SMEM)] + [vspec] * 15,
        out_specs=vspec,
    )(agent, state, deg_parts, c_parts, Wc, bc, W1, b1, g1, bt1,
      W2, b2, g2, bt2, Wmu, bmu)


def kernel(state, edge_index, agent_i, W_conv, b_conv, W1, b1, g1, beta1,
           W2, b2, g2, beta2, Wmu, bmu):
    ei = edge_index.astype(jnp.int32)
    agent_vec = jnp.full((L,), agent_i, dtype=jnp.int32)
    deg_parts, c_parts = _sc_hist(ei, agent_vec)

    agent = jnp.asarray(agent_i, jnp.int32).reshape(1)
    out = _tc_head(agent, state, deg_parts, c_parts,
                   W_conv, b_conv.reshape(1, HID),
                   W1, b1.reshape(1, 256), g1.reshape(1, 256), beta1.reshape(1, 256),
                   W2, b2.reshape(1, 128), g2.reshape(1, 128), beta2.reshape(1, 128),
                   Wmu, bmu.reshape(1, 64))
    return out.reshape(64)


# speedup vs baseline: 298.8887x; 1.1323x over previous
import jax
import jax.numpy as jnp
from jax import lax
from jax.experimental import pallas as pl
from jax.experimental.pallas import tpu as pltpu
from jax.experimental.pallas import tpu_sc as plsc

N = 10000
E = 320000
D_IN = 128
HID = 256
NW = 32
E_PER_W = E // NW
L = 16


_BLKS = E // 128
_B_LO = _BLKS // NW
_B_EXTRA = _BLKS % NW
_CHUNK_MAX = (_B_LO + 1) * 128


def _sc_hist_body(edge_hbm, agent_hbm, deg_out, c_out,
                  ev, agent_v, deg_v, c_v, sem):
    wid = lax.axis_index("s") * 2 + lax.axis_index("c")

    nblk = _B_LO + jnp.where(wid < _B_EXTRA, 1, 0)
    col0 = wid * (_B_LO * 128) + jnp.minimum(wid, _B_EXTRA) * 128
    ncol = nblk * 128

    edge_dma = pltpu.async_copy(
        edge_hbm.at[:, pl.ds(col0, ncol)], ev.at[:, pl.ds(0, ncol)], sem)
    pltpu.sync_copy(agent_hbm, agent_v)

    zero = jnp.zeros((L,), jnp.float32)

    def _zero(i, _):
        deg_v[pl.ds(i * L, L)] = zero
        c_v[pl.ds(i * L, L)] = zero
        return 0

    lax.fori_loop(0, N // L, _zero, 0, unroll=8)
    edge_dma.wait()

    agent = agent_v[...]
    ones = jnp.ones((L,), jnp.float32)

    def _scan(i):
        s = ev[0, pl.ds(i, L)]
        d = ev[1, pl.ds(i, L)]
        plsc.addupdate_scatter(deg_v, [d], ones)
        plsc.addupdate_scatter(c_v, [s], ones, mask=d == agent)

    plsc.parallel_loop(0, _B_LO * 128, step=L, unroll=8)(_scan)

    @pl.when(wid < _B_EXTRA)
    def _extra():
        plsc.parallel_loop(_B_LO * 128, (_B_LO + 1) * 128, step=L,
                           unroll=8)(_scan)

    pltpu.sync_copy(deg_v, deg_out.at[wid])
    pltpu.sync_copy(c_v, c_out.at[wid])


_sc_hist = pl.kernel(
    _sc_hist_body,
    out_type=(
        jax.ShapeDtypeStruct((NW, N), jnp.float32),
        jax.ShapeDtypeStruct((NW, N), jnp.float32),
    ),
    mesh=plsc.VectorSubcoreMesh(core_axis_name="c", subcore_axis_name="s"),
    compiler_params=pltpu.CompilerParams(needs_layout_passes=False),
    scratch_types=[
        pltpu.VMEM((2, _CHUNK_MAX), jnp.int32),
        pltpu.VMEM((L,), jnp.int32),
        pltpu.VMEM((N,), jnp.float32),
        pltpu.VMEM((N,), jnp.float32),
        pltpu.SemaphoreType.DMA,
    ],
)


def _tc_head_body(agent_ref, state_ref, degp_ref, cp_ref,
                  Wc_ref, bc_ref, W1_ref, b1_ref, g1_ref, bt1_ref,
                  W2_ref, b2_ref, g2_ref, bt2_ref, Wmu_ref, bmu_ref, out_ref):
    a = agent_ref[0]
    deg = jnp.sum(degp_ref[...], axis=0, keepdims=True) + 1.0
    c = jnp.sum(cp_ref[...], axis=0, keepdims=True)
    dinv = lax.rsqrt(deg)
    col = lax.broadcasted_iota(jnp.int32, (1, N), 1)
    is_a = col == a
    da = jnp.sum(jnp.where(is_a, dinv, 0.0))
    u = da * (c * dinv) + (da * da) * jnp.where(is_a, 1.0, 0.0)

    acc = jnp.dot(u, state_ref[...], preferred_element_type=jnp.float32)
    x = jnp.dot(acc, Wc_ref[...], preferred_element_type=jnp.float32) + bc_ref[...]
    x = jnp.maximum(x, 0.0)

    x = jnp.dot(x, W1_ref[...], preferred_element_type=jnp.float32) + b1_ref[...]
    m = jnp.mean(x, axis=-1, keepdims=True)
    v = jnp.mean((x - m) ** 2, axis=-1, keepdims=True)
    x = (x - m) * lax.rsqrt(v + 1e-5) * g1_ref[...] + bt1_ref[...]
    x = jnp.maximum(x, 0.0)

    x = jnp.dot(x, W2_ref[...], preferred_element_type=jnp.float32) + b2_ref[...]
    m = jnp.mean(x, axis=-1, keepdims=True)
    v = jnp.mean((x - m) ** 2, axis=-1, keepdims=True)
    x = (x - m) * lax.rsqrt(v + 1e-5) * g2_ref[...] + bt2_ref[...]
    x = jnp.maximum(x, 0.0)

    x = jnp.dot(x, Wmu_ref[...], preferred_element_type=jnp.float32) + bmu_ref[...]
    out_ref[...] = jax.nn.sigmoid(x)


def _tc_head(agent, state, deg_parts, c_parts, Wc, bc, W1, b1, g1, bt1,
             W2, b2, g2, bt2, Wmu, bmu):
    vspec = pl.BlockSpec()
    return pl.pallas_call(
        _tc_head_body,
        out_shape=jax.ShapeDtypeStruct((1, 64), jnp.float32),
        in_specs=[pl.BlockSpec(memory_space=pltpu.SMEM)] + [vspec] * 15,
        out_specs=vspec,
    )(agent, state, deg_parts, c_parts, Wc, bc, W1, b1, g1, bt1,
      W2, b2, g2, bt2, Wmu, bmu)


def kernel(state, edge_index, agent_i, W_conv, b_conv, W1, b1, g1, beta1,
           W2, b2, g2, beta2, Wmu, bmu):
    ei = edge_index.astype(jnp.int32)
    agent_vec = jnp.full((L,), agent_i, dtype=jnp.int32)
    deg_parts, c_parts = _sc_hist(ei, agent_vec)

    agent = jnp.asarray(agent_i, jnp.int32).reshape(1)
    out = _tc_head(agent, state, deg_parts, c_parts,
                   W_conv, b_conv.reshape(1, HID),
                   W1, b1.reshape(1, 256), g1.reshape(1, 256), beta1.reshape(1, 256),
                   W2, b2.reshape(1, 128), g2.reshape(1, 128), beta2.reshape(1, 128),
                   Wmu, bmu.reshape(1, 64))
    return out.reshape(64)
